# baseline (speedup 1.0000x reference)
import jax
import jax.numpy as jnp
from jax import lax
from jax.experimental import pallas as pl
from jax.experimental.pallas import tpu as pltpu
from jax.experimental.pallas import tpu_sc as plsc

VOCAB = 100000
OBJ = 100000
DIM = 200
N_NEG = 100

LANES = 16
N_COL_TILES = 7
N_COLS_PAD = N_COL_TILES * LANES
CHUNK = 128
N_CHUNKS = DIM * LANES // CHUNK
WROW_PAD = 208
WCHUNK = 104


def _rsqrt(x):
    i = plsc.bitcast(x, jnp.int32)
    i = jnp.int32(0x5F3759DF) - lax.shift_right_logical(i, 1)
    y = plsc.bitcast(i, jnp.float32)
    for _ in range(3):
        y = y * (1.5 - 0.5 * x * y * y)
    return y


def _sc_body(oflat, wflat, oidx, wrepidx, wlinidx, out_res, out_w,
             oidx_v, wrepidx_v, wlinidx_v, col_v, wrep_v, wrow_v, res_v,
             sem_o, sem_w, sem_l):
    wid = lax.axis_index("s") * 2 + lax.axis_index("c")

    @pl.when(wid < N_COL_TILES)
    def _():
        pltpu.sync_copy(oidx.at[wid], oidx_v)
        pltpu.sync_copy(wrepidx, wrepidx_v)

        def fire(g, carry):
            pltpu.async_copy(oflat.at[oidx_v.at[g]],
                             col_v.at[pl.ds(g * CHUNK, CHUNK)], sem_o)
            pltpu.async_copy(wflat.at[wrepidx_v.at[g]],
                             wrep_v.at[pl.ds(g * CHUNK, CHUNK)], sem_w)
            return carry
        lax.fori_loop(0, N_CHUNKS, fire, 0)

        @pl.when(wid == 0)
        def _():
            pltpu.sync_copy(wlinidx, wlinidx_v)
            for g in range(2):
                pltpu.async_copy(wflat.at[wlinidx_v.at[g]],
                                 wrow_v.at[pl.ds(g * WCHUNK, WCHUNK)], sem_l)

        pltpu.make_async_copy(oflat.at[pl.ds(0, DIM * LANES)], col_v, sem_o).wait()
        pltpu.make_async_copy(wflat.at[pl.ds(0, DIM * LANES)], wrep_v, sem_w).wait()

        def body(d, carry):
            nacc, sacc, wsacc = carry
            col = col_v[pl.ds(d * LANES, LANES)]
            ws = wrep_v[pl.ds(d * LANES, LANES)]
            return (nacc + ws * col, sacc + col * col, wsacc + ws * ws)
        zeros = jnp.zeros((LANES,), jnp.float32)
        nacc, sacc, wsacc = lax.fori_loop(0, DIM, body, (zeros, zeros, zeros))

        eps2 = jnp.float32(1e-16)
        denom2 = jnp.maximum(wsacc, eps2) * jnp.maximum(sacc, eps2)
        res_v[...] = nacc * _rsqrt(denom2)
        pltpu.sync_copy(res_v, out_res.at[pl.ds(wid * LANES, LANES)])

        @pl.when(wid == 0)
        def _():
            pltpu.make_async_copy(wflat.at[pl.ds(0, WROW_PAD)], wrow_v, sem_l).wait()
            pltpu.sync_copy(wrow_v.at[pl.ds(0, DIM)], out_w)


_sc_call = pl.kernel(
    _sc_body,
    out_type=(
        jax.ShapeDtypeStruct((N_COLS_PAD,), jnp.float32),
        jax.ShapeDtypeStruct((DIM,), jnp.float32),
    ),
    mesh=plsc.VectorSubcoreMesh(core_axis_name="c", subcore_axis_name="s",
                                num_cores=2, num_subcores=16),
    compiler_params=pltpu.CompilerParams(needs_layout_passes=False),
    scratch_types=(
        pltpu.VMEM((N_CHUNKS, CHUNK), jnp.int32),
        pltpu.VMEM((N_CHUNKS, CHUNK), jnp.int32),
        pltpu.VMEM((2, WCHUNK), jnp.int32),
        pltpu.VMEM((DIM * LANES,), jnp.float32),
        pltpu.VMEM((DIM * LANES,), jnp.float32),
        pltpu.VMEM((WROW_PAD,), jnp.float32),
        pltpu.VMEM((LANES,), jnp.float32),
        pltpu.SemaphoreType.DMA,
        pltpu.SemaphoreType.DMA,
        pltpu.SemaphoreType.DMA,
    ),
)


def kernel(W, O, word, obj, neg_samples):
    word = jnp.asarray(word, jnp.int32)
    obj = jnp.asarray(obj, jnp.int32)
    neg = jnp.asarray(neg_samples, jnp.int32)

    col_idx = jnp.concatenate(
        [obj.reshape(1), neg, jnp.zeros((N_COLS_PAD - 1 - N_NEG,), jnp.int32)])
    d_ar = jnp.arange(DIM, dtype=jnp.int32)
    oidx = (d_ar[None, :, None] * OBJ
            + col_idx.reshape(N_COL_TILES, 1, LANES)).reshape(
                N_COL_TILES, N_CHUNKS, CHUNK)
    wrepidx = jnp.broadcast_to((word * DIM + d_ar)[:, None],
                               (DIM, LANES)).reshape(N_CHUNKS, CHUNK)
    wlinidx = (word * DIM
               + jnp.minimum(jnp.arange(WROW_PAD, dtype=jnp.int32), DIM - 1)
               ).reshape(2, WCHUNK)

    res_all, wrow = _sc_call(O.reshape(-1), W.reshape(-1),
                             oidx, wrepidx, wlinidx)
    word_embed = wrow.reshape(1, DIM)
    obj_embed = res_all[0]
    neg_embeds = res_all[1:1 + N_NEG]
    return (word_embed, obj_embed, neg_embeds)


# speedup vs baseline: 4.4343x; 4.4343x over previous
import jax
import jax.numpy as jnp
from jax import lax
from jax.experimental import pallas as pl
from jax.experimental.pallas import tpu as pltpu

VOCAB = 100000
OBJ = 100000
DIM = 200
N_NEG = 100
N_IDX = N_NEG + 1
LANE = 128


def _tc_body(cols, word, o_blk, w_blk, res, wout):
    i = pl.program_id(0)
    lane = lax.rem(cols[i], LANE)

    o = o_blk[...]
    w = w_blk[pl.ds(lax.rem(word[0], 8), 1), :]
    num_full = jnp.dot(w, o, preferred_element_type=jnp.float32,
                       precision=lax.Precision.HIGHEST)
    sq_full = jnp.sum(o * o, axis=0, keepdims=True)
    wsq = jnp.sum(w * w)

    eps2 = jnp.float32(1e-16)
    denom2 = jnp.maximum(wsq, eps2) * jnp.maximum(sq_full, eps2)
    r_vec = num_full * lax.rsqrt(denom2)

    lane_iota = lax.broadcasted_iota(jnp.int32, (1, LANE), 1)
    r_scalar = jnp.sum(jnp.where(lane_iota == lane, r_vec, 0.0))

    @pl.when(i == 0)
    def _():
        res[...] = jnp.zeros((1, LANE), jnp.float32)
        wout[...] = w
    res[...] = jnp.where(lane_iota == i, r_scalar, res[...])


_grid_spec = pltpu.PrefetchScalarGridSpec(
    num_scalar_prefetch=2,
    grid=(N_IDX,),
    in_specs=[
        pl.BlockSpec((DIM, LANE), lambda i, cols, word: (0, cols[i] // LANE)),
        pl.BlockSpec((8, DIM), lambda i, cols, word: (word[0] // 8, 0)),
    ],
    out_specs=[
        pl.BlockSpec((1, LANE), lambda i, cols, word: (0, 0)),
        pl.BlockSpec((1, DIM), lambda i, cols, word: (0, 0)),
    ],
)

_tc_call = pl.pallas_call(
    _tc_body,
    grid_spec=_grid_spec,
    out_shape=(
        jax.ShapeDtypeStruct((1, LANE), jnp.float32),
        jax.ShapeDtypeStruct((1, DIM), jnp.float32),
    ),
)


def kernel(W, O, word, obj, neg_samples):
    word = jnp.asarray(word, jnp.int32).reshape(1)
    obj = jnp.asarray(obj, jnp.int32)
    neg = jnp.asarray(neg_samples, jnp.int32)
    cols = jnp.concatenate([obj.reshape(1), neg])

    res, wout = _tc_call(cols, word, O, W)
    word_embed = wout
    obj_embed = res[0, 0]
    neg_embeds = res[0, 1:1 + N_NEG]
    return (word_embed, obj_embed, neg_embeds)
